# baseline (speedup 1.0000x reference)
import functools

import jax
import jax.numpy as jnp
from jax import lax
from jax.experimental import pallas as pl
from jax.experimental.pallas import tpu as pltpu
from jax.experimental.pallas import tpu_sc as plsc

B = 16384
D = 64
L = 16
NC, NS = 2, 16
NW = NC * NS
BPW = B // NW
GCH = 128
NCHUNK = BPW // GCH
NG = BPW // L

def _sc_fwd_impl(uid_hbm, iid_hbm, uemb_hbm, iemb_hbm, wext_hbm, out_hbm,
                 uidx_v, iidx_v, urows, irows, wv, outv, sem):
    wid = lax.axis_index("s") * NC + lax.axis_index("c")
    base = wid * BPW

    pltpu.sync_copy(uid_hbm.at[wid], uidx_v)
    pltpu.sync_copy(iid_hbm.at[wid], iidx_v)
    pltpu.sync_copy(wext_hbm, wv)

    copies = []
    for j in range(NCHUNK):
        copies.append(pltpu.async_copy(
            uemb_hbm.at[uidx_v.at[j]], urows.at[pl.ds(j * GCH, GCH)], sem))
        copies.append(pltpu.async_copy(
            iemb_hbm.at[iidx_v.at[j]], irows.at[pl.ds(j * GCH, GCH)], sem))
    for c in copies:
        c.wait()

    wchunks = [wv[pl.ds(16 * j, L)] for j in range(8)]
    bias = wv[pl.ds(128, L)]
    iota = lax.iota(jnp.int32, L)

    def dg(v, idx):
        return lax.gather(
            v, idx.reshape(L, 1),
            dimension_numbers=lax.GatherDimensionNumbers(
                offset_dims=(), collapsed_slice_dims=(0,), start_index_map=(0,)),
            slice_sizes=(1,),
            mode=lax.GatherScatterMode.PROMISE_IN_BOUNDS,
        )

    brev = [int(f"{s:04b}"[::-1], 2) for s in range(L)]

    def group(g, _):
        vecs = []
        for s in range(L):
            row = g * L + brev[s]
            t = urows[row, pl.ds(0, L)] * wchunks[0]
            for j in range(1, 4):
                t = t + urows[row, pl.ds(16 * j, L)] * wchunks[j]
            for j in range(4):
                t = t + irows[row, pl.ds(16 * j, L)] * wchunks[4 + j]
            vecs.append(t)
        for h in (8, 4, 2, 1):
            folded = [v + dg(v, iota ^ h) for v in vecs]
            vecs = [
                jnp.where((iota & h) == 0, folded[2 * p], folded[2 * p + 1])
                for p in range(len(folded) // 2)
            ]
        outv[pl.ds(g * L, L)] = vecs[0] + bias
        return _

    lax.fori_loop(0, NG, group, 0)
    pltpu.sync_copy(outv, out_hbm.at[pl.ds(base, BPW)])


@functools.cache
def _build_sc_fwd():
    mesh = plsc.VectorSubcoreMesh(
        core_axis_name="c", subcore_axis_name="s",
        num_cores=NC, num_subcores=NS,
    )
    return pl.kernel(
        _sc_fwd_impl,
        out_type=jax.ShapeDtypeStruct((B,), jnp.float32),
        mesh=mesh,
        scratch_types=[
            pltpu.VMEM((NCHUNK, GCH), jnp.int32),
            pltpu.VMEM((NCHUNK, GCH), jnp.int32),
            pltpu.VMEM((BPW, D), jnp.float32),
            pltpu.VMEM((BPW, D), jnp.float32),
            pltpu.VMEM((144,), jnp.float32),
            pltpu.VMEM((BPW,), jnp.float32),
            pltpu.SemaphoreType.DMA,
        ],
        compiler_params=pltpu.CompilerParams(use_tc_tiling_on_sc=False),
    )


def kernel(train_x, u_emb, i_emb, W, b):
    uid = train_x[:, 0].reshape(NW, NCHUNK, GCH)
    iid = train_x[:, 1].reshape(NW, NCHUNK, GCH)
    wext = jnp.concatenate(
        [W.reshape(-1), jnp.broadcast_to(b.reshape(-1)[0], (L,))]
    ).astype(jnp.float32)
    return _build_sc_fwd()(uid, iid, u_emb, i_emb, wext)
